# baseline (speedup 1.0000x reference)
import functools

import jax
import jax.numpy as jnp
from jax import lax
from jax.experimental import pallas as pl
from jax.experimental.pallas import tpu as pltpu
from jax.experimental.pallas import tpu_sc as plsc

_NUM_ROWS = 16384
_NUM_COLS = 1000
_NUM_CORES = 2
_NUM_SUBCORES = 16
_NW = _NUM_CORES * _NUM_SUBCORES
_ROWS_PER_W = _NUM_ROWS // _NW
_CHUNK = 32
_NCH = _ROWS_PER_W // _CHUNK
_CHUNK_ELEMS = _CHUNK * _NUM_COLS


def _sc_onehot(idx_hbm, zeros_hbm, out_hbm, buf0, buf1, idx_v, sem0, sem1):
    wid = lax.axis_index("s") * _NUM_CORES + lax.axis_index("c")
    base_row = wid * _ROWS_PER_W

    pltpu.sync_copy(idx_hbm.at[pl.ds(base_row * 1, _ROWS_PER_W)], idx_v)
    pltpu.sync_copy(zeros_hbm, buf0)
    pltpu.sync_copy(zeros_hbm, buf1)

    iota16 = lax.iota(jnp.int32, 16)
    ones16 = jnp.full((16,), 1.0, jnp.float32)
    zerosf = jnp.zeros((16,), jnp.float32)
    bufs = (buf0, buf1)
    sems = (sem0, sem1)

    def positions(c, g):
        col = idx_v[pl.ds(c * _CHUNK + g * 16, 16)]
        return (iota16 + g * 16) * _NUM_COLS + col

    copies = [None, None]
    for c in range(_NCH):
        buf = bufs[c % 2]
        sem = sems[c % 2]
        if c >= 2:
            copies[c % 2].wait()
            for g in range(_CHUNK // 16):
                plsc.store_scatter(buf, [positions(c - 2, g)], zerosf)
        for g in range(_CHUNK // 16):
            plsc.store_scatter(buf, [positions(c, g)], ones16)
        dst = out_hbm.at[pl.ds((base_row + c * _CHUNK) * _NUM_COLS, _CHUNK_ELEMS)]
        copies[c % 2] = pltpu.async_copy(buf, dst, sem)
    copies[0].wait()
    copies[1].wait()


def kernel(x, device, num_classes):
    n = x.shape[0]
    rk = jax.random.key(42)
    pred_ints = jax.random.randint(rk, (n,), 0, num_classes).astype(jnp.int32)
    zeros_chunk = jnp.zeros((_CHUNK_ELEMS,), jnp.float32)

    mesh = plsc.VectorSubcoreMesh(core_axis_name="c", subcore_axis_name="s")
    run = pl.kernel(
        _sc_onehot,
        out_type=jax.ShapeDtypeStruct((n * _NUM_COLS,), jnp.float32),
        mesh=mesh,
        compiler_params=pltpu.CompilerParams(needs_layout_passes=False),
        scratch_types=[
            pltpu.VMEM((_CHUNK_ELEMS,), jnp.float32),
            pltpu.VMEM((_CHUNK_ELEMS,), jnp.float32),
            pltpu.VMEM((_ROWS_PER_W,), jnp.int32),
            pltpu.SemaphoreType.DMA,
            pltpu.SemaphoreType.DMA,
        ],
    )
    out = run(pred_ints, zeros_chunk)
    return out.reshape(n, _NUM_COLS)


# speedup vs baseline: 1.4955x; 1.4955x over previous
import jax
import jax.numpy as jnp
from jax import lax
from jax.experimental import pallas as pl
from jax.experimental.pallas import tpu as pltpu
from jax.experimental.pallas import tpu_sc as plsc

_NUM_ROWS = 16384
_NUM_COLS = 1000
_NUM_CORES = 2
_NUM_SUBCORES = 16
_NW = _NUM_CORES * _NUM_SUBCORES
_ROWS_PER_W = _NUM_ROWS // _NW
_CHUNK = 32
_NCH = _ROWS_PER_W // _CHUNK


def _sc_onehot(idx_hbm, zeros_hbm, out_hbm, buf0, buf1, idx_v, sem0, sem1):
    wid = lax.axis_index("s") * _NUM_CORES + lax.axis_index("c")
    base_row = wid * _ROWS_PER_W

    pltpu.sync_copy(idx_hbm.at[pl.ds(base_row, _ROWS_PER_W)], idx_v)
    pltpu.sync_copy(zeros_hbm, buf0)
    pltpu.sync_copy(zeros_hbm, buf1)

    iota16 = lax.iota(jnp.int32, 16)
    ones16 = jnp.full((16,), 1.0, jnp.float32)
    zerosf = jnp.zeros((16,), jnp.float32)
    bufs = (buf0, buf1)
    sems = (sem0, sem1)

    def indices(c, g):
        col = idx_v[pl.ds(c * _CHUNK + g * 16, 16)]
        row = iota16 + g * 16
        return row, col

    copies = [None, None]
    for c in range(_NCH):
        buf = bufs[c % 2]
        sem = sems[c % 2]
        if c >= 2:
            copies[c % 2].wait()
            for g in range(_CHUNK // 16):
                row, col = indices(c - 2, g)
                plsc.store_scatter(buf, [row, col], zerosf)
        for g in range(_CHUNK // 16):
            row, col = indices(c, g)
            plsc.store_scatter(buf, [row, col], ones16)
        dst = out_hbm.at[pl.ds((base_row + c * _CHUNK), _CHUNK), :]
        copies[c % 2] = pltpu.async_copy(buf, dst, sem)
    copies[0].wait()
    copies[1].wait()


def kernel(x, device, num_classes):
    n = x.shape[0]
    rk = jax.random.key(42)
    pred_ints = jax.random.randint(rk, (n,), 0, num_classes).astype(jnp.int32)
    zeros_chunk = jnp.zeros((_CHUNK, _NUM_COLS), jnp.float32)

    mesh = plsc.VectorSubcoreMesh(core_axis_name="c", subcore_axis_name="s")
    run = pl.kernel(
        _sc_onehot,
        out_type=jax.ShapeDtypeStruct((n, _NUM_COLS), jnp.float32),
        mesh=mesh,
        compiler_params=pltpu.CompilerParams(
            needs_layout_passes=False,
            use_tc_tiling_on_sc=True,
        ),
        scratch_types=[
            pltpu.VMEM((_CHUNK, _NUM_COLS), jnp.float32),
            pltpu.VMEM((_CHUNK, _NUM_COLS), jnp.float32),
            pltpu.VMEM((_ROWS_PER_W,), jnp.int32),
            pltpu.SemaphoreType.DMA,
            pltpu.SemaphoreType.DMA,
        ],
    )
    return run(pred_ints, zeros_chunk)


# speedup vs baseline: 5.6677x; 3.7899x over previous
import jax
import jax.numpy as jnp
from jax import lax
from jax.experimental import pallas as pl

_NUM_ROWS = 16384
_NUM_COLS = 1000
_COL_BLOCK = 2048


def _onehot_t_block(idx_ref, out_ref):
    classes = lax.broadcasted_iota(jnp.int32, out_ref.shape, 0)
    out_ref[...] = (classes == idx_ref[0]).astype(jnp.float32)


def kernel(x, device, num_classes):
    n = x.shape[0]
    rk = jax.random.key(42)
    pred_ints = jax.random.randint(rk, (n,), 0, num_classes)
    idx3 = pred_ints.astype(jnp.int32).reshape(n // _COL_BLOCK, 1, _COL_BLOCK)
    out_t = pl.pallas_call(
        _onehot_t_block,
        grid=(n // _COL_BLOCK,),
        in_specs=[pl.BlockSpec((1, 1, _COL_BLOCK), lambda i: (i, 0, 0))],
        out_specs=pl.BlockSpec((_NUM_COLS, _COL_BLOCK), lambda i: (0, i)),
        out_shape=jax.ShapeDtypeStruct((_NUM_COLS, n), jnp.float32),
    )(idx3)
    return out_t.T
